# baseline (speedup 1.0000x reference)
import jax
import jax.numpy as jnp
from jax import lax
from jax.experimental import pallas as pl
from jax.experimental.pallas import tpu as pltpu
from jax.experimental.pallas import tpu_sc as plsc

N = 10000
E = 320000
D = 128
NPAD = 10240
NC = 2
NS = 16
NW = NC * NS
EW = E // NW
EWP = 10240
PADW = EWP - EW
K = 128
NR = EWP // K // 8
RPC = NPAD // NS


def _z16():
    return jnp.zeros((16,), jnp.float32)


def _deg_body(src_hbm, dst_hbm, dego_hbm, degi_hbm, cs_hbm,
              sbuf, dbuf, dov, div, csv, red, outv, spm):
    c = lax.axis_index("c")
    s = lax.axis_index("s")
    wid = s * NC + c

    def zero(i, _):
        dov[pl.ds(i * 16, 16)] = _z16()
        div[pl.ds(i * 16, 16)] = _z16()
        csv[pl.ds(i * 16, 16)] = _z16()
    lax.fori_loop(0, NPAD // 16, zero, None)

    pltpu.sync_copy(src_hbm.at[pl.ds(wid * EW, EW)], sbuf)
    pltpu.sync_copy(dst_hbm.at[pl.ds(wid * EW, EW)], dbuf)

    ones16 = jnp.ones((16,), jnp.float32)

    def count(i, _):
        sv = sbuf[pl.ds(i * 16, 16)]
        dv = dbuf[pl.ds(i * 16, 16)]
        m = sv != dv
        plsc.addupdate_scatter(dov, [sv], ones16, mask=m)
        plsc.addupdate_scatter(div, [dv], ones16, mask=m)
        plsc.addupdate_scatter(csv, [sv], ones16, mask=jnp.logical_not(m))
    lax.fori_loop(0, EW // 16, count, None)

    pltpu.sync_copy(dov, spm.at[0, s])
    pltpu.sync_copy(div, spm.at[1, s])
    pltpu.sync_copy(csv, spm.at[2, s])
    plsc.subcore_barrier()

    for a, out_hbm in ((0, dego_hbm), (1, degi_hbm), (2, cs_hbm)):
        pltpu.sync_copy(spm.at[a, :, pl.ds(s * RPC, RPC)], red)

        def reduce(j, _):
            accv = _z16()
            for r in range(NS):
                accv = accv + red[r, pl.ds(j * 16, 16)]
            outv[pl.ds(j * 16, 16)] = accv
        lax.fori_loop(0, RPC // 16, reduce, None)
        pltpu.sync_copy(outv, out_hbm.at[c, pl.ds(s * RPC, RPC)])


def _sc_degrees(src, dst):
    return pl.kernel(
        _deg_body,
        out_type=[jax.ShapeDtypeStruct((NC, NPAD), jnp.float32)] * 3,
        mesh=plsc.VectorSubcoreMesh(core_axis_name="c", subcore_axis_name="s"),
        scratch_types=[
            pltpu.VMEM((EW,), jnp.int32),
            pltpu.VMEM((EW,), jnp.int32),
            pltpu.VMEM((NPAD,), jnp.float32),
            pltpu.VMEM((NPAD,), jnp.float32),
            pltpu.VMEM((NPAD,), jnp.float32),
            pltpu.VMEM((NS, RPC), jnp.float32),
            pltpu.VMEM((RPC,), jnp.float32),
            pltpu.VMEM_SHARED((3, NS, NPAD), jnp.float32),
        ],
        compiler_params=pltpu.CompilerParams(needs_layout_passes=False),
    )(src, dst)


def _agg_body(h_hbm, srcm_hbm, dstm_hbm, acc_hbm,
              bs0, bd0, bs1, bd1, rows0, rows1, zb, spm,
              semg0, semg1, semi0, semi1):
    c = lax.axis_index("c")
    s = lax.axis_index("s")
    wid = s * NC + c

    def zero(i, _):
        for j in range(8):
            zb[i, pl.ds(j * 16, 16)] = _z16()
    lax.fori_loop(0, 32, zero, None)

    def zacc(t, _):
        pltpu.sync_copy(zb, spm.at[pl.ds(s * RPC + t * 32, 32), :])
    lax.fori_loop(0, RPC // 32, zacc, None)
    plsc.subcore_barrier()

    bigs = ((bs0, bd0, semi0), (bs1, bd1, semi1))
    rowbufs = ((rows0, semg0), (rows1, semg1))

    def stage(t2, slot):
        bs, bd, semi = bigs[slot]
        pltpu.async_copy(srcm_hbm.at[wid, pl.ds(t2 * 8, 8), :], bs, semi)
        pltpu.async_copy(dstm_hbm.at[wid, pl.ds(t2 * 8, 8), :], bd, semi)

    def wait_stage(t2, slot):
        bs, bd, semi = bigs[slot]
        pltpu.make_async_copy(srcm_hbm.at[wid, pl.ds(t2 * 8, 8), :],
                              bs, semi).wait()
        pltpu.make_async_copy(dstm_hbm.at[wid, pl.ds(t2 * 8, 8), :],
                              bd, semi).wait()

    def fire_g(si, slot):
        rows, semg = rowbufs[slot]
        pltpu.async_copy(h_hbm.at[si], rows, semg)

    def wait_g(si, slot):
        rows, semg = rowbufs[slot]
        pltpu.make_async_copy(h_hbm.at[si], rows, semg).wait()

    stage(0, 0)
    stage(1, 1)
    wait_stage(0, 0)
    fire_g(bs0.at[0], 0)

    def titer(t, _):
        for j in range(16):
            half, q = (0, j) if j < 8 else (1, j - 8)
            jn = j + 1
            if jn < 16:
                hn, qn = (0, jn) if jn < 8 else (1, jn - 8)
                if jn == 8:
                    wait_stage(2 * t + 1, 1)
                fire_g(bigs[hn][0].at[qn], jn % 2)
            else:
                @pl.when(t < NR // 2 - 1)
                def _():
                    wait_stage(2 * t + 2, 0)
                    fire_g(bs0.at[0], 0)
            wait_g(bigs[half][0].at[q], j % 2)
            rows = rowbufs[j % 2][0]
            pltpu.sync_copy(rows, spm.at[bigs[half][1].at[q]], add=True)
            if j == 7:
                @pl.when(t < NR // 2 - 1)
                def _():
                    stage(2 * t + 2, 0)
            if j == 15:
                @pl.when(t < NR // 2 - 1)
                def _():
                    stage(2 * t + 3, 1)
    lax.fori_loop(0, NR // 2, titer, None)

    plsc.subcore_barrier()

    def epi(t, _):
        r0 = s * RPC + t * 32
        pltpu.sync_copy(spm.at[pl.ds(r0, 32), :], zb)
        pltpu.sync_copy(zb, acc_hbm.at[c, pl.ds(r0, 32), :])
    lax.fori_loop(0, RPC // 32, epi, None)


def _sc_aggregate(h, srcm, dstm):
    return pl.kernel(
        _agg_body,
        out_type=jax.ShapeDtypeStruct((NC, NPAD, D), jnp.float32),
        mesh=plsc.VectorSubcoreMesh(core_axis_name="c", subcore_axis_name="s"),
        scratch_types=[
            pltpu.VMEM((8, K), jnp.int32),
            pltpu.VMEM((8, K), jnp.int32),
            pltpu.VMEM((8, K), jnp.int32),
            pltpu.VMEM((8, K), jnp.int32),
            pltpu.VMEM((K, D), jnp.float32),
            pltpu.VMEM((K, D), jnp.float32),
            pltpu.VMEM((32, D), jnp.float32),
            pltpu.VMEM_SHARED((NPAD, D), jnp.float32),
            pltpu.SemaphoreType.DMA,
            pltpu.SemaphoreType.DMA,
            pltpu.SemaphoreType.DMA,
            pltpu.SemaphoreType.DMA,
        ],
        compiler_params=pltpu.CompilerParams(needs_layout_passes=False),
    )(h, srcm, dstm)


def _scale_body(nf_ref, dego_ref, h_ref):
    deg = dego_ref[0] + dego_ref[1] + 1.0
    h_ref[...] = nf_ref[...] * lax.rsqrt(deg)


def _tc_scale(nf, dego3):
    rb = 1000
    return pl.pallas_call(
        _scale_body,
        grid=(N // rb,),
        in_specs=[
            pl.BlockSpec((rb, D), lambda i: (i, 0)),
            pl.BlockSpec((NC, rb, 1), lambda i: (0, i, 0)),
        ],
        out_specs=pl.BlockSpec((rb, D), lambda i: (i, 0)),
        out_shape=jax.ShapeDtypeStruct((N, D), jnp.float32),
    )(nf, dego3)


def _out_body(acc_ref, h_ref, degi_ref, cs_ref, w_ref, b_ref, o_ref):
    rd = h_ref.shape[0]
    ridx = lax.broadcasted_iota(jnp.int32, (rd, 1), 0) + pl.program_id(0) * rd
    pad_ind = (ridx < NW * PADW).astype(jnp.float32)
    hterm = (1.0 - cs_ref[0] - cs_ref[1] - pad_ind) * h_ref[...]
    x = acc_ref[0] + acc_ref[1] + hterm
    nrm = lax.rsqrt(degi_ref[0] + degi_ref[1] + 1.0)
    x = x * nrm
    o_ref[...] = (
        jnp.dot(x, w_ref[...], preferred_element_type=jnp.float32) + b_ref[...]
    )


def _tc_out(acc, h, degi3, cs3, W, b2):
    rd = 1000
    return pl.pallas_call(
        _out_body,
        grid=(N // rd,),
        in_specs=[
            pl.BlockSpec((NC, rd, D), lambda i: (0, i, 0)),
            pl.BlockSpec((rd, D), lambda i: (i, 0)),
            pl.BlockSpec((NC, rd, 1), lambda i: (0, i, 0)),
            pl.BlockSpec((NC, rd, 1), lambda i: (0, i, 0)),
            pl.BlockSpec((D, D), lambda i: (0, 0)),
            pl.BlockSpec((1, D), lambda i: (0, 0)),
        ],
        out_specs=pl.BlockSpec((rd, D), lambda i: (i, 0)),
        out_shape=jax.ShapeDtypeStruct((N, D), jnp.float32),
    )(acc, h, degi3, cs3, W, b2)


def kernel(nf, edge_index, W, b):
    src = edge_index[0]
    dst = edge_index[1]

    dego, degi, cs = _sc_degrees(src, dst)
    h = _tc_scale(nf, dego.reshape(NC, NPAD, 1))

    pad_v = (jnp.arange(NW * PADW, dtype=jnp.int32) % N).reshape(NW, PADW)
    srcm = jnp.concatenate([src.reshape(NW, EW), pad_v], axis=1)
    dstm = jnp.concatenate([dst.reshape(NW, EW), pad_v], axis=1)
    srcm = srcm.reshape(NW, EWP // K, K)
    dstm = dstm.reshape(NW, EWP // K, K)

    acc = _sc_aggregate(h, srcm, dstm)
    out = _tc_out(acc, h, degi.reshape(NC, NPAD, 1),
                  cs.reshape(NC, NPAD, 1), W, b.reshape(1, D))
    return out


# speedup vs baseline: 1.0182x; 1.0182x over previous
import jax
import jax.numpy as jnp
from jax import lax
from jax.experimental import pallas as pl
from jax.experimental.pallas import tpu as pltpu
from jax.experimental.pallas import tpu_sc as plsc

N = 10000
E = 320000
D = 128
NPAD = 10240
NC = 2
NS = 16
NW = NC * NS
EW = E // NW
EWP = 10240
PADW = EWP - EW
K = 128
NR = EWP // K // 8
RPC = NPAD // NS


def _z16():
    return jnp.zeros((16,), jnp.float32)


def _deg_body(src_hbm, dst_hbm, dego_hbm, degi_hbm, cs_hbm,
              sbuf, dbuf, dov, div, csv, red, outv, spm, sems):
    c = lax.axis_index("c")
    s = lax.axis_index("s")
    wid = s * NC + c

    pltpu.async_copy(src_hbm.at[pl.ds(wid * EW, EW)], sbuf, sems)
    pltpu.async_copy(dst_hbm.at[pl.ds(wid * EW, EW)], dbuf, sems)

    def zero(i, _):
        dov[pl.ds(i * 16, 16)] = _z16()
        div[pl.ds(i * 16, 16)] = _z16()
        csv[pl.ds(i * 16, 16)] = _z16()
    lax.fori_loop(0, NPAD // 16, zero, None)

    pltpu.make_async_copy(src_hbm.at[pl.ds(wid * EW, EW)], sbuf, sems).wait()
    pltpu.make_async_copy(dst_hbm.at[pl.ds(wid * EW, EW)], dbuf, sems).wait()

    ones16 = jnp.ones((16,), jnp.float32)

    def count(i, _):
        sv = sbuf[pl.ds(i * 16, 16)]
        dv = dbuf[pl.ds(i * 16, 16)]
        m = sv != dv
        plsc.addupdate_scatter(dov, [sv], ones16, mask=m)
        plsc.addupdate_scatter(div, [dv], ones16, mask=m)
        plsc.addupdate_scatter(csv, [sv], ones16, mask=jnp.logical_not(m))
    lax.fori_loop(0, EW // 16, count, None)

    pltpu.sync_copy(dov, spm.at[0, s])
    pltpu.sync_copy(div, spm.at[1, s])
    pltpu.sync_copy(csv, spm.at[2, s])
    plsc.subcore_barrier()

    for a, out_hbm in ((0, dego_hbm), (1, degi_hbm), (2, cs_hbm)):
        pltpu.sync_copy(spm.at[a, :, pl.ds(s * RPC, RPC)], red)

        def reduce(j, _):
            accv = _z16()
            for r in range(NS):
                accv = accv + red[r, pl.ds(j * 16, 16)]
            outv[pl.ds(j * 16, 16)] = accv
        lax.fori_loop(0, RPC // 16, reduce, None)
        pltpu.sync_copy(outv, out_hbm.at[c, pl.ds(s * RPC, RPC)])


def _sc_degrees(src, dst):
    return pl.kernel(
        _deg_body,
        out_type=[jax.ShapeDtypeStruct((NC, NPAD), jnp.float32)] * 3,
        mesh=plsc.VectorSubcoreMesh(core_axis_name="c", subcore_axis_name="s"),
        scratch_types=[
            pltpu.VMEM((EW,), jnp.int32),
            pltpu.VMEM((EW,), jnp.int32),
            pltpu.VMEM((NPAD,), jnp.float32),
            pltpu.VMEM((NPAD,), jnp.float32),
            pltpu.VMEM((NPAD,), jnp.float32),
            pltpu.VMEM((NS, RPC), jnp.float32),
            pltpu.VMEM((RPC,), jnp.float32),
            pltpu.VMEM_SHARED((3, NS, NPAD), jnp.float32),
            pltpu.SemaphoreType.DMA,
        ],
        compiler_params=pltpu.CompilerParams(needs_layout_passes=False),
    )(src, dst)


def _agg_body(h_hbm, srcm_hbm, dstm_hbm, acc_hbm,
              bs0, bd0, bs1, bd1, rows0, rows1, zb, spm,
              semg0, semg1, semi0, semi1):
    c = lax.axis_index("c")
    s = lax.axis_index("s")
    wid = s * NC + c

    bigs = ((bs0, bd0, semi0), (bs1, bd1, semi1))
    rowbufs = ((rows0, semg0), (rows1, semg1))

    def stage(t2, slot):
        bs, bd, semi = bigs[slot]
        pltpu.async_copy(srcm_hbm.at[wid, pl.ds(t2 * 8, 8), :], bs, semi)
        pltpu.async_copy(dstm_hbm.at[wid, pl.ds(t2 * 8, 8), :], bd, semi)

    stage(0, 0)
    stage(1, 1)

    def zero(i, _):
        for j in range(8):
            zb[i, pl.ds(j * 16, 16)] = _z16()
    lax.fori_loop(0, 32, zero, None)

    def zacc(t, _):
        pltpu.sync_copy(zb, spm.at[pl.ds(s * RPC + t * 32, 32), :])
    lax.fori_loop(0, RPC // 32, zacc, None)
    plsc.subcore_barrier()

    def wait_stage(t2, slot):
        bs, bd, semi = bigs[slot]
        pltpu.make_async_copy(srcm_hbm.at[wid, pl.ds(t2 * 8, 8), :],
                              bs, semi).wait()
        pltpu.make_async_copy(dstm_hbm.at[wid, pl.ds(t2 * 8, 8), :],
                              bd, semi).wait()

    def fire_g(si, slot):
        rows, semg = rowbufs[slot]
        pltpu.async_copy(h_hbm.at[si], rows, semg)

    def wait_g(si, slot):
        rows, semg = rowbufs[slot]
        pltpu.make_async_copy(h_hbm.at[si], rows, semg).wait()

    wait_stage(0, 0)
    fire_g(bs0.at[0], 0)

    def titer(t, _):
        for j in range(16):
            half, q = (0, j) if j < 8 else (1, j - 8)
            jn = j + 1
            if jn < 16:
                hn, qn = (0, jn) if jn < 8 else (1, jn - 8)
                if jn == 8:
                    wait_stage(2 * t + 1, 1)
                fire_g(bigs[hn][0].at[qn], jn % 2)
            else:
                @pl.when(t < NR // 2 - 1)
                def _():
                    wait_stage(2 * t + 2, 0)
                    fire_g(bs0.at[0], 0)
            wait_g(bigs[half][0].at[q], j % 2)
            rows = rowbufs[j % 2][0]
            pltpu.sync_copy(rows, spm.at[bigs[half][1].at[q]], add=True)
            if j == 7:
                @pl.when(t < NR // 2 - 1)
                def _():
                    stage(2 * t + 2, 0)
            if j == 15:
                @pl.when(t < NR // 2 - 1)
                def _():
                    stage(2 * t + 3, 1)
    lax.fori_loop(0, NR // 2, titer, None)

    plsc.subcore_barrier()

    def epi(t, _):
        r0 = s * RPC + t * 32
        pltpu.sync_copy(spm.at[pl.ds(r0, 32), :], zb)
        pltpu.sync_copy(zb, acc_hbm.at[c, pl.ds(r0, 32), :])
    lax.fori_loop(0, RPC // 32, epi, None)


def _sc_aggregate(h, srcm, dstm):
    return pl.kernel(
        _agg_body,
        out_type=jax.ShapeDtypeStruct((NC, NPAD, D), jnp.float32),
        mesh=plsc.VectorSubcoreMesh(core_axis_name="c", subcore_axis_name="s"),
        scratch_types=[
            pltpu.VMEM((8, K), jnp.int32),
            pltpu.VMEM((8, K), jnp.int32),
            pltpu.VMEM((8, K), jnp.int32),
            pltpu.VMEM((8, K), jnp.int32),
            pltpu.VMEM((K, D), jnp.float32),
            pltpu.VMEM((K, D), jnp.float32),
            pltpu.VMEM((32, D), jnp.float32),
            pltpu.VMEM_SHARED((NPAD, D), jnp.float32),
            pltpu.SemaphoreType.DMA,
            pltpu.SemaphoreType.DMA,
            pltpu.SemaphoreType.DMA,
            pltpu.SemaphoreType.DMA,
        ],
        compiler_params=pltpu.CompilerParams(needs_layout_passes=False),
    )(h, srcm, dstm)


def _scale_body(nf_ref, dego_ref, h_ref):
    deg = dego_ref[0] + dego_ref[1] + 1.0
    h_ref[...] = nf_ref[...] * lax.rsqrt(deg)


def _tc_scale(nf, dego3):
    rb = 1000
    return pl.pallas_call(
        _scale_body,
        grid=(N // rb,),
        in_specs=[
            pl.BlockSpec((rb, D), lambda i: (i, 0)),
            pl.BlockSpec((NC, rb, 1), lambda i: (0, i, 0)),
        ],
        out_specs=pl.BlockSpec((rb, D), lambda i: (i, 0)),
        out_shape=jax.ShapeDtypeStruct((N, D), jnp.float32),
    )(nf, dego3)


def _out_body(acc_ref, h_ref, degi_ref, cs_ref, w_ref, b_ref, o_ref):
    rd = h_ref.shape[0]
    ridx = lax.broadcasted_iota(jnp.int32, (rd, 1), 0) + pl.program_id(0) * rd
    pad_ind = (ridx < NW * PADW).astype(jnp.float32)
    hterm = (1.0 - cs_ref[0] - cs_ref[1] - pad_ind) * h_ref[...]
    x = acc_ref[0] + acc_ref[1] + hterm
    nrm = lax.rsqrt(degi_ref[0] + degi_ref[1] + 1.0)
    x = x * nrm
    o_ref[...] = (
        jnp.dot(x, w_ref[...], preferred_element_type=jnp.float32) + b_ref[...]
    )


def _tc_out(acc, h, degi3, cs3, W, b2):
    rd = 1000
    return pl.pallas_call(
        _out_body,
        grid=(N // rd,),
        in_specs=[
            pl.BlockSpec((NC, rd, D), lambda i: (0, i, 0)),
            pl.BlockSpec((rd, D), lambda i: (i, 0)),
            pl.BlockSpec((NC, rd, 1), lambda i: (0, i, 0)),
            pl.BlockSpec((NC, rd, 1), lambda i: (0, i, 0)),
            pl.BlockSpec((D, D), lambda i: (0, 0)),
            pl.BlockSpec((1, D), lambda i: (0, 0)),
        ],
        out_specs=pl.BlockSpec((rd, D), lambda i: (i, 0)),
        out_shape=jax.ShapeDtypeStruct((N, D), jnp.float32),
    )(acc, h, degi3, cs3, W, b2)


def kernel(nf, edge_index, W, b):
    src = edge_index[0]
    dst = edge_index[1]

    dego, degi, cs = _sc_degrees(src, dst)
    h = _tc_scale(nf, dego.reshape(NC, NPAD, 1))

    pad_v = (jnp.arange(NW * PADW, dtype=jnp.int32) % N).reshape(NW, PADW)
    srcm = jnp.concatenate([src.reshape(NW, EW), pad_v], axis=1)
    dstm = jnp.concatenate([dst.reshape(NW, EW), pad_v], axis=1)
    srcm = srcm.reshape(NW, EWP // K, K)
    dstm = dstm.reshape(NW, EWP // K, K)

    acc = _sc_aggregate(h, srcm, dstm)
    out = _tc_out(acc, h, degi.reshape(NC, NPAD, 1),
                  cs.reshape(NC, NPAD, 1), W, b.reshape(1, D))
    return out
